# baseline (speedup 1.0000x reference)
import functools
import math

import jax
import jax.numpy as jnp
from jax import lax
from jax.experimental import pallas as pl
from jax.experimental.pallas import tpu as pltpu
from jax.experimental.pallas import tpu_sc as plsc

ALPHA = 0.5
THETA = 1.0

_NC = 2
_NS = 16
_NW = _NC * _NS
_K = 112
_G = 8
_ZR = 40


def _deg_body(N, nc, dst_hbm, ones_hbm, zer_hbm, out_hbm, idx_d, onev, zbuf, acc):
    c = lax.axis_index("c")
    s = lax.axis_index("s")
    wid = c * _NS + s
    nch = N // _ZR

    pltpu.sync_copy(dst_hbm.at[wid], idx_d)
    pltpu.sync_copy(ones_hbm, onev)
    pltpu.sync_copy(zer_hbm, zbuf)

    def zc_body(m, carry):
        ch = m * _NS + s

        @pl.when(ch < nch)
        def _():
            pltpu.sync_copy(zbuf, acc.at[pl.ds(ch * _ZR, _ZR)])

        return carry

    lax.fori_loop(0, pl.cdiv(nch, _NS), zc_body, 0)
    plsc.subcore_barrier()

    def body(j, carry):
        pltpu.sync_copy(onev, acc.at[idx_d.at[j]], add=True)
        return carry

    lax.fori_loop(0, nc, body, 0)
    plsc.subcore_barrier()

    def out_body(m, carry):
        ch = m * _NS + s

        @pl.when(ch < nch)
        def _():
            pltpu.sync_copy(acc.at[pl.ds(ch * _ZR, _ZR)], zbuf)
            pltpu.sync_copy(zbuf, out_hbm.at[c, pl.ds(ch * _ZR, _ZR)])

        return carry

    lax.fori_loop(0, pl.cdiv(nch, _NS), out_body, 0)


def _make_deg_kernel(N, D, nc):
    mesh = plsc.VectorSubcoreMesh(core_axis_name="c", subcore_axis_name="s")
    return pl.kernel(
        functools.partial(_deg_body, N, nc),
        mesh=mesh,
        out_type=jax.ShapeDtypeStruct((_NC, N, D), jnp.float32),
        scratch_types=[
            pltpu.VMEM((nc, _K), jnp.int32),
            pltpu.VMEM((_K, D), jnp.float32),
            pltpu.VMEM((_ZR, D), jnp.float32),
            pltpu.VMEM_SHARED((N + 8, D), jnp.float32),
        ],
    )


def _agg_body(N, nc, g_hbm, src_hbm, dst_hbm, zer_hbm, out_hbm,
              idx_s, dg0, dg1, rows0, rows1, zbuf, acc,
              sg0, sg1, sd0, sd1):
    c = lax.axis_index("c")
    s = lax.axis_index("s")
    wid = c * _NS + s
    nfull = nc // _G
    assert nc % _G == 0 and nfull % 2 == 0

    pltpu.sync_copy(src_hbm.at[wid], idx_s)
    pltpu.async_copy(dst_hbm.at[wid, pl.ds(0, _G)], dg0, sd0)
    pltpu.async_copy(dst_hbm.at[wid, pl.ds(_G, _G)], dg1, sd1)

    pltpu.sync_copy(zer_hbm, zbuf)
    nch = N // _ZR

    def zc_body(m, carry):
        ch = m * _NS + s

        @pl.when(ch < nch)
        def _():
            pltpu.sync_copy(zbuf, acc.at[pl.ds(ch * _ZR, _ZR)])

        return carry

    lax.fori_loop(0, pl.cdiv(nch, _NS), zc_body, 0)

    @pl.when(s == 0)
    def _():
        pltpu.sync_copy(zer_hbm.at[pl.ds(0, 8)], acc.at[pl.ds(N, 8)])

    plsc.subcore_barrier()

    pltpu.async_copy(g_hbm.at[idx_s.at[0]], rows0, sg0)
    pltpu.async_copy(g_hbm.at[idx_s.at[1]], rows1, sg1)

    def run_group(gbase, dg, sd, next_base):
        pltpu.make_async_copy(dst_hbm.at[wid, pl.ds(0, _G)], dg, sd).wait()

        def pair(l2, carry):
            for t, (rb, sgb) in ((0, (rows0, sg0)), (1, (rows1, sg1))):
                l = 2 * l2 + t
                j = gbase + l
                pltpu.make_async_copy(g_hbm.at[idx_s.at[0]], rb, sgb).wait()
                pltpu.sync_copy(rb, acc.at[dg.at[l]], add=True)

                @pl.when(j + 2 < nc)
                def _():
                    pltpu.async_copy(g_hbm.at[idx_s.at[j + 2]], rb, sgb)

            return carry

        lax.fori_loop(0, _G // 2, pair, 0)

        @pl.when(next_base < nc)
        def _():
            pltpu.async_copy(dst_hbm.at[wid, pl.ds(next_base, _G)], dg, sd)

    def groups(m, carry):
        gbase = (2 * m) * _G
        run_group(gbase, dg0, sd0, gbase + 2 * _G)
        run_group(gbase + _G, dg1, sd1, gbase + 3 * _G)
        return carry

    lax.fori_loop(0, nfull // 2, groups, 0)
    plsc.subcore_barrier()

    def out_body(m, carry):
        ch = m * _NS + s

        @pl.when(ch < nch)
        def _():
            pltpu.sync_copy(acc.at[pl.ds(ch * _ZR, _ZR)], zbuf)
            pltpu.sync_copy(zbuf, out_hbm.at[c, pl.ds(ch * _ZR, _ZR)])

        return carry

    lax.fori_loop(0, pl.cdiv(nch, _NS), out_body, 0)


def _make_agg_kernel(N, D, nc):
    mesh = plsc.VectorSubcoreMesh(core_axis_name="c", subcore_axis_name="s")
    return pl.kernel(
        functools.partial(_agg_body, N, nc),
        mesh=mesh,
        out_type=jax.ShapeDtypeStruct((_NC, N, D), jnp.float32),
        scratch_types=[
            pltpu.VMEM((nc, _K), jnp.int32),
            pltpu.VMEM((_G, _K), jnp.int32),
            pltpu.VMEM((_G, _K), jnp.int32),
            pltpu.VMEM((_K, D), jnp.float32),
            pltpu.VMEM((_K, D), jnp.float32),
            pltpu.VMEM((_ZR, D), jnp.float32),
            pltpu.VMEM_SHARED((N + 8, D), jnp.float32),
            pltpu.SemaphoreType.DMA,
            pltpu.SemaphoreType.DMA,
            pltpu.SemaphoreType.DMA,
            pltpu.SemaphoreType.DMA,
        ],
    )


def _w2_body(betas, x_ref, w2_ref, xw2_ref):
    x0 = jnp.maximum(x_ref[...], 0.0)
    for i, b in enumerate(betas):
        m = jnp.dot(x0, w2_ref[i], preferred_element_type=jnp.float32)
        xw2_ref[i] = ALPHA * ((1.0 - b) * x0 + b * m)


def _make_w2(N, D, Lw, betas, R):
    return pl.pallas_call(
        functools.partial(_w2_body, betas),
        grid=(N // R,),
        in_specs=[
            pl.BlockSpec((R, D), lambda i: (i, 0)),
            pl.BlockSpec((Lw, D, D), lambda i: (0, 0, 0)),
        ],
        out_specs=[pl.BlockSpec((Lw, R, D), lambda i: (0, i, 0))],
        out_shape=[jax.ShapeDtypeStruct((Lw, N, D), jnp.float32)],
    )


def _pre_body(x_ref, degp_ref, g0_ref, dinv_ref):
    x0 = jnp.maximum(x_ref[...], 0.0)
    deg = degp_ref[0][:, 0:1] + degp_ref[1][:, 0:1] + 1.0
    dinv = lax.rsqrt(deg)
    dinv_ref[...] = jnp.broadcast_to(dinv, x0.shape)
    g0_ref[...] = dinv * x0


def _make_pre(N, D, R):
    return pl.pallas_call(
        _pre_body,
        grid=(N // R,),
        in_specs=[
            pl.BlockSpec((R, D), lambda i: (i, 0)),
            pl.BlockSpec((_NC, R, D), lambda i: (0, i, 0)),
        ],
        out_specs=[
            pl.BlockSpec((R, D), lambda i: (i, 0)),
            pl.BlockSpec((R, D), lambda i: (i, 0)),
        ],
        out_shape=[
            jax.ShapeDtypeStruct((N, D), jnp.float32),
            jax.ShapeDtypeStruct((N, D), jnp.float32),
        ],
    )


def _layer_body(beta, p_ref, g_ref, dinv_ref, xw2_ref, w1_ref, h_ref, g2_ref):
    dinv = dinv_ref[...]
    t = dinv * (p_ref[0] + p_ref[1] + g_ref[...])
    m = jnp.dot(t, w1_ref[...], preferred_element_type=jnp.float32)
    u = (1.0 - ALPHA) * ((1.0 - beta) * t + beta * m) + xw2_ref[...]
    h = jnp.maximum(u, 0.0)
    h_ref[...] = h
    g2_ref[...] = dinv * h


def _make_layer(N, D, beta, R):
    return pl.pallas_call(
        functools.partial(_layer_body, beta),
        grid=(N // R,),
        in_specs=[
            pl.BlockSpec((_NC, R, D), lambda i: (0, i, 0)),
            pl.BlockSpec((R, D), lambda i: (i, 0)),
            pl.BlockSpec((R, D), lambda i: (i, 0)),
            pl.BlockSpec((R, D), lambda i: (i, 0)),
            pl.BlockSpec((D, D), lambda i: (0, 0)),
        ],
        out_specs=[
            pl.BlockSpec((R, D), lambda i: (i, 0)),
            pl.BlockSpec((R, D), lambda i: (i, 0)),
        ],
        out_shape=[
            jax.ShapeDtypeStruct((N, D), jnp.float32),
            jax.ShapeDtypeStruct((N, D), jnp.float32),
        ],
    )


@jax.jit
def kernel(x, edge_index, W1, W2):
    N, D = x.shape
    E = edge_index.shape[1]
    Lw = W1.shape[0]
    betas = [math.log(THETA / (i + 1) + 1.0) for i in range(Lw)]
    R = 1000

    E_w = E // _NW
    nc = pl.cdiv(pl.cdiv(E_w, _K), _G) * _G
    pad_w = nc * _K - E_w
    dpad = N + (jnp.arange(pad_w, dtype=jnp.int32) % 8)
    src3d = jnp.concatenate(
        [edge_index[0].reshape(_NW, E_w),
         jnp.zeros((_NW, pad_w), jnp.int32)], axis=1).reshape(_NW, nc, _K)
    dst3d = jnp.concatenate(
        [edge_index[1].reshape(_NW, E_w),
         jnp.broadcast_to(dpad, (_NW, pad_w))], axis=1).reshape(_NW, nc, _K)
    zer = jnp.zeros((_ZR, D), jnp.float32)
    onesk = jnp.ones((_K, D), jnp.float32)

    degp = _make_deg_kernel(N, D, nc)(dst3d, onesk, zer)
    (xw2,) = _make_w2(N, D, Lw, betas, R)(x, W2)
    g0, dinvb = _make_pre(N, D, R)(x, degp)

    agg = _make_agg_kernel(N, D, nc)
    g = g0
    h = None
    for i in range(Lw):
        part = agg(g, src3d, dst3d, zer)
        h, g = _make_layer(N, D, betas[i], R)(part, g, dinvb, xw2[i], W1[i])
    return h


# speedup vs baseline: 2.4117x; 2.4117x over previous
import functools
import math

import jax
import jax.numpy as jnp
from jax import lax
from jax.experimental import pallas as pl
from jax.experimental.pallas import tpu as pltpu
from jax.experimental.pallas import tpu_sc as plsc

ALPHA = 0.5
THETA = 1.0

_NC = 2
_NS = 16
_NW = _NC * _NS
_K = 128
_G = 8
_ZR = 40


def _deg_body(N, nc, dst_hbm, ones_hbm, zer_hbm, out_hbm, idx_d, onev, zbuf, acc):
    c = lax.axis_index("c")
    s = lax.axis_index("s")
    wid = c * _NS + s
    nch = N // _ZR

    pltpu.sync_copy(dst_hbm.at[wid], idx_d)
    pltpu.sync_copy(ones_hbm, onev)
    pltpu.sync_copy(zer_hbm, zbuf)

    def zc_body(m, carry):
        ch = m * _NS + s

        @pl.when(ch < nch)
        def _():
            pltpu.sync_copy(zbuf, acc.at[pl.ds(ch * _ZR, _ZR)])

        return carry

    lax.fori_loop(0, pl.cdiv(nch, _NS), zc_body, 0)
    plsc.subcore_barrier()

    def body(j, carry):
        pltpu.sync_copy(onev, acc.at[idx_d.at[j]], add=True)
        return carry

    lax.fori_loop(0, nc, body, 0)
    plsc.subcore_barrier()

    def out_body(m, carry):
        ch = m * _NS + s

        @pl.when(ch < nch)
        def _():
            pltpu.sync_copy(acc.at[pl.ds(ch * _ZR, _ZR)], zbuf)
            pltpu.sync_copy(zbuf, out_hbm.at[c, pl.ds(ch * _ZR, _ZR)])

        return carry

    lax.fori_loop(0, pl.cdiv(nch, _NS), out_body, 0)


def _make_deg_kernel(N, D, nc, n_acc):
    mesh = plsc.VectorSubcoreMesh(core_axis_name="c", subcore_axis_name="s")
    return pl.kernel(
        functools.partial(_deg_body, N, nc),
        mesh=mesh,
        out_type=jax.ShapeDtypeStruct((_NC, N, D), jnp.float32),
        scratch_types=[
            pltpu.VMEM((nc, _K), jnp.int32),
            pltpu.VMEM((_K, D), jnp.float32),
            pltpu.VMEM((_ZR, D), jnp.float32),
            pltpu.VMEM_SHARED((n_acc, D), jnp.float32),
        ],
    )


def _agg_body(N, nc, g_hbm, src_hbm, dst_hbm, zer_hbm, out_hbm,
              idx_s, dg0, dg1, rows0, rows1, acc,
              sg0, sg1, sd0, sd1):
    c = lax.axis_index("c")
    s = lax.axis_index("s")
    wid = c * _NS + s
    nfull = nc // _G
    assert nc % _G == 0 and nfull % 2 == 0

    pltpu.sync_copy(src_hbm.at[wid], idx_s)
    pltpu.async_copy(dst_hbm.at[wid, pl.ds(0, _G)], dg0, sd0)
    pltpu.async_copy(dst_hbm.at[wid, pl.ds(_G, _G)], dg1, sd1)

    pltpu.sync_copy(zer_hbm, rows0.at[pl.ds(0, _ZR)])
    nch = N // _ZR

    def zc_body(m, carry):
        ch = m * _NS + s

        @pl.when(ch < nch)
        def _():
            pltpu.sync_copy(rows0.at[pl.ds(0, _ZR)], acc.at[pl.ds(ch * _ZR, _ZR)])

        return carry

    lax.fori_loop(0, pl.cdiv(nch, _NS), zc_body, 0)
    plsc.subcore_barrier()

    pltpu.async_copy(g_hbm.at[idx_s.at[0]], rows0, sg0)
    pltpu.async_copy(g_hbm.at[idx_s.at[1]], rows1, sg1)

    def run_group(gbase, dg, sd, next_base):
        pltpu.make_async_copy(dst_hbm.at[wid, pl.ds(0, _G)], dg, sd).wait()

        def pair(l2, carry):
            for t, (rb, sgb) in ((0, (rows0, sg0)), (1, (rows1, sg1))):
                l = 2 * l2 + t
                j = gbase + l
                pltpu.make_async_copy(g_hbm.at[idx_s.at[0]], rb, sgb).wait()
                pltpu.sync_copy(rb, acc.at[dg.at[l]], add=True)

                @pl.when(j + 2 < nc)
                def _():
                    pltpu.async_copy(g_hbm.at[idx_s.at[j + 2]], rb, sgb)

            return carry

        lax.fori_loop(0, _G // 2, pair, 0)

        @pl.when(next_base < nc)
        def _():
            pltpu.async_copy(dst_hbm.at[wid, pl.ds(next_base, _G)], dg, sd)

    def groups(m, carry):
        gbase = (2 * m) * _G
        run_group(gbase, dg0, sd0, gbase + 2 * _G)
        run_group(gbase + _G, dg1, sd1, gbase + 3 * _G)
        return carry

    lax.fori_loop(0, nfull // 2, groups, 0)
    plsc.subcore_barrier()

    def out_body(m, carry):
        ch = m * _NS + s

        @pl.when(ch < nch)
        def _():
            pltpu.sync_copy(acc.at[pl.ds(ch * _ZR, _ZR)], rows0.at[pl.ds(0, _ZR)])
            pltpu.sync_copy(rows0.at[pl.ds(0, _ZR)], out_hbm.at[c, pl.ds(ch * _ZR, _ZR)])

        return carry

    lax.fori_loop(0, pl.cdiv(nch, _NS), out_body, 0)


def _make_agg_kernel(N, D, nc, n_acc):
    mesh = plsc.VectorSubcoreMesh(core_axis_name="c", subcore_axis_name="s")
    return pl.kernel(
        functools.partial(_agg_body, N, nc),
        mesh=mesh,
        out_type=jax.ShapeDtypeStruct((_NC, N, D), jnp.float32),
        scratch_types=[
            pltpu.VMEM((nc, _K), jnp.int32),
            pltpu.VMEM((_G, _K), jnp.int32),
            pltpu.VMEM((_G, _K), jnp.int32),
            pltpu.VMEM((_K, D), jnp.float32),
            pltpu.VMEM((_K, D), jnp.float32),
            pltpu.VMEM_SHARED((n_acc, D), jnp.float32),
            pltpu.SemaphoreType.DMA,
            pltpu.SemaphoreType.DMA,
            pltpu.SemaphoreType.DMA,
            pltpu.SemaphoreType.DMA,
        ],
    )


def _w2_body(betas, x_ref, w2_ref, xw2_ref):
    x0 = jnp.maximum(x_ref[...], 0.0)
    for i, b in enumerate(betas):
        m = jnp.dot(x0, w2_ref[i], preferred_element_type=jnp.float32)
        xw2_ref[i] = ALPHA * ((1.0 - b) * x0 + b * m)


def _make_w2(N, D, Lw, betas, R):
    return pl.pallas_call(
        functools.partial(_w2_body, betas),
        grid=(N // R,),
        in_specs=[
            pl.BlockSpec((R, D), lambda i: (i, 0)),
            pl.BlockSpec((Lw, D, D), lambda i: (0, 0, 0)),
        ],
        out_specs=[pl.BlockSpec((Lw, R, D), lambda i: (0, i, 0))],
        out_shape=[jax.ShapeDtypeStruct((Lw, N, D), jnp.float32)],
    )


def _pre_body(x_ref, degp_ref, g0_ref, dinv_ref):
    x0 = jnp.maximum(x_ref[...], 0.0)
    deg = degp_ref[0][:, 0:1] + degp_ref[1][:, 0:1] + 1.0
    dinv = lax.rsqrt(deg)
    dinv_ref[...] = jnp.broadcast_to(dinv, x0.shape)
    g0_ref[...] = dinv * x0


def _make_pre(N, D, R):
    return pl.pallas_call(
        _pre_body,
        grid=(N // R,),
        in_specs=[
            pl.BlockSpec((R, D), lambda i: (i, 0)),
            pl.BlockSpec((_NC, R, D), lambda i: (0, i, 0)),
        ],
        out_specs=[
            pl.BlockSpec((R, D), lambda i: (i, 0)),
            pl.BlockSpec((R, D), lambda i: (i, 0)),
        ],
        out_shape=[
            jax.ShapeDtypeStruct((N, D), jnp.float32),
            jax.ShapeDtypeStruct((N, D), jnp.float32),
        ],
    )


def _layer_body(beta, p_ref, g_ref, dinv_ref, xw2_ref, w1_ref, h_ref, g2_ref):
    dinv = dinv_ref[...]
    t = dinv * (p_ref[0] + p_ref[1] + g_ref[...])
    m = jnp.dot(t, w1_ref[...], preferred_element_type=jnp.float32)
    u = (1.0 - ALPHA) * ((1.0 - beta) * t + beta * m) + xw2_ref[...]
    h = jnp.maximum(u, 0.0)
    h_ref[...] = h
    g2_ref[...] = dinv * h


def _make_layer(N, D, beta, R):
    return pl.pallas_call(
        functools.partial(_layer_body, beta),
        grid=(N // R,),
        in_specs=[
            pl.BlockSpec((_NC, R, D), lambda i: (0, i, 0)),
            pl.BlockSpec((R, D), lambda i: (i, 0)),
            pl.BlockSpec((R, D), lambda i: (i, 0)),
            pl.BlockSpec((R, D), lambda i: (i, 0)),
            pl.BlockSpec((D, D), lambda i: (0, 0)),
        ],
        out_specs=[
            pl.BlockSpec((R, D), lambda i: (i, 0)),
            pl.BlockSpec((R, D), lambda i: (i, 0)),
        ],
        out_shape=[
            jax.ShapeDtypeStruct((N, D), jnp.float32),
            jax.ShapeDtypeStruct((N, D), jnp.float32),
        ],
    )


@jax.jit
def kernel(x, edge_index, W1, W2):
    N, D = x.shape
    E = edge_index.shape[1]
    Lw = W1.shape[0]
    betas = [math.log(THETA / (i + 1) + 1.0) for i in range(Lw)]
    R = 1000

    E_w = E // _NW
    nc = pl.cdiv(pl.cdiv(E_w, _K), _G) * _G
    pad_w = nc * _K - E_w
    dpad = N + jnp.arange(pad_w, dtype=jnp.int32)
    src3d = jnp.concatenate(
        [edge_index[0].reshape(_NW, E_w),
         jnp.zeros((_NW, pad_w), jnp.int32)], axis=1).reshape(_NW, nc, _K)
    dst3d = jnp.concatenate(
        [edge_index[1].reshape(_NW, E_w),
         jnp.broadcast_to(dpad, (_NW, pad_w))], axis=1).reshape(_NW, nc, _K)
    zer = jnp.zeros((_ZR, D), jnp.float32)
    onesk = jnp.ones((_K, D), jnp.float32)

    n_acc = N + ((pad_w + 7) // 8) * 8
    degp = _make_deg_kernel(N, D, nc, n_acc)(dst3d, onesk, zer)
    (xw2,) = _make_w2(N, D, Lw, betas, R)(x, W2)
    g0, dinvb = _make_pre(N, D, R)(x, degp)

    agg = _make_agg_kernel(N, D, nc, n_acc)
    g = g0
    h = None
    for i in range(Lw):
        part = agg(g, src3d, dst3d, zer)
        h, g = _make_layer(N, D, betas[i], R)(part, g, dinvb, xw2[i], W1[i])
    return h


# speedup vs baseline: 6.6518x; 2.7582x over previous
import functools
import math

import jax
import jax.numpy as jnp
from jax import lax
from jax.experimental import pallas as pl
from jax.experimental.pallas import tpu as pltpu
from jax.experimental.pallas import tpu_sc as plsc

ALPHA = 0.5
THETA = 1.0

_NC = 2
_NS = 16
_NW = _NC * _NS
_K = 100
_G = 16
_ZR = 40


def _deg_body(N, nc, dst_hbm, ones_hbm, zer_hbm, out_hbm, idx_d, onev, zbuf, acc):
    c = lax.axis_index("c")
    s = lax.axis_index("s")
    wid = c * _NS + s
    nch = N // _ZR

    pltpu.sync_copy(dst_hbm.at[wid], idx_d)
    pltpu.sync_copy(ones_hbm, onev)
    pltpu.sync_copy(zer_hbm, zbuf)

    def zc_body(m, carry):
        ch = m * _NS + s

        @pl.when(ch < nch)
        def _():
            pltpu.sync_copy(zbuf, acc.at[pl.ds(ch * _ZR, _ZR)])

        return carry

    lax.fori_loop(0, pl.cdiv(nch, _NS), zc_body, 0)
    plsc.subcore_barrier()

    def body(j, carry):
        pltpu.sync_copy(onev, acc.at[idx_d.at[j]], add=True)
        return carry

    lax.fori_loop(0, nc, body, 0)
    plsc.subcore_barrier()

    def out_body(m, carry):
        ch = m * _NS + s

        @pl.when(ch < nch)
        def _():
            pltpu.sync_copy(acc.at[pl.ds(ch * _ZR, _ZR)], zbuf)
            pltpu.sync_copy(zbuf, out_hbm.at[c, pl.ds(ch * _ZR, _ZR)])

        return carry

    lax.fori_loop(0, pl.cdiv(nch, _NS), out_body, 0)


def _make_deg_kernel(N, D, nc, nc_pad):
    mesh = plsc.VectorSubcoreMesh(core_axis_name="c", subcore_axis_name="s")
    return pl.kernel(
        functools.partial(_deg_body, N, nc),
        mesh=mesh,
        out_type=jax.ShapeDtypeStruct((_NC, N, D), jnp.float32),
        scratch_types=[
            pltpu.VMEM((nc_pad, _K), jnp.int32),
            pltpu.VMEM((_K, D), jnp.float32),
            pltpu.VMEM((_ZR, D), jnp.float32),
            pltpu.VMEM_SHARED((N, D), jnp.float32),
        ],
    )


def _agg_body(N, nc, g_hbm, src_hbm, dst_hbm, zer_hbm, out_hbm,
              idx_s, dg0, dg1, rows0, rows1, zbuf, acc,
              sg0, sg1, sd0, sd1):
    c = lax.axis_index("c")
    s = lax.axis_index("s")
    wid = c * _NS + s
    nfull = nc // _G
    rem = nc - nfull * _G
    assert nfull % 2 == 0 and rem % 2 == 0 and rem < _G

    pltpu.sync_copy(src_hbm.at[wid], idx_s)
    pltpu.async_copy(dst_hbm.at[wid, pl.ds(0, _G)], dg0, sd0)
    pltpu.async_copy(dst_hbm.at[wid, pl.ds(_G, _G)], dg1, sd1)

    pltpu.sync_copy(zer_hbm, zbuf)
    nch = N // _ZR

    def zc_body(m, carry):
        ch = m * _NS + s

        @pl.when(ch < nch)
        def _():
            pltpu.sync_copy(zbuf, acc.at[pl.ds(ch * _ZR, _ZR)])

        return carry

    lax.fori_loop(0, pl.cdiv(nch, _NS), zc_body, 0)
    plsc.subcore_barrier()

    pltpu.async_copy(g_hbm.at[idx_s.at[0]], rows0, sg0)
    pltpu.async_copy(g_hbm.at[idx_s.at[1]], rows1, sg1)

    def run_group(gbase, dg, sd, next_base):
        pltpu.make_async_copy(dst_hbm.at[wid, pl.ds(0, _G)], dg, sd).wait()

        def pair(l2, carry):
            for t, (rb, sgb) in ((0, (rows0, sg0)), (1, (rows1, sg1))):
                l = 2 * l2 + t
                j = gbase + l
                pltpu.make_async_copy(g_hbm.at[idx_s.at[0]], rb, sgb).wait()
                pltpu.sync_copy(rb, acc.at[dg.at[l]], add=True)

                @pl.when(j + 2 < nc)
                def _():
                    pltpu.async_copy(g_hbm.at[idx_s.at[j + 2]], rb, sgb)

            return carry

        lax.fori_loop(0, _G // 2, pair, 0)

        @pl.when(next_base < nc)
        def _():
            pltpu.async_copy(dst_hbm.at[wid, pl.ds(next_base, _G)], dg, sd)

    def groups(m, carry):
        gbase = (2 * m) * _G
        run_group(gbase, dg0, sd0, gbase + 2 * _G)
        run_group(gbase + _G, dg1, sd1, gbase + 3 * _G)
        return carry

    lax.fori_loop(0, nfull // 2, groups, 0)

    if rem:
        pltpu.make_async_copy(dst_hbm.at[wid, pl.ds(0, _G)], dg0, sd0).wait()
        for l in range(rem):
            j = nfull * _G + l
            rb, sgb = (rows0, sg0) if l % 2 == 0 else (rows1, sg1)
            pltpu.make_async_copy(g_hbm.at[idx_s.at[0]], rb, sgb).wait()
            pltpu.sync_copy(rb, acc.at[dg0.at[l]], add=True)
            if l + 2 < rem:
                pltpu.async_copy(g_hbm.at[idx_s.at[j + 2]], rb, sgb)
    plsc.subcore_barrier()

    def out_body(m, carry):
        ch = m * _NS + s

        @pl.when(ch < nch)
        def _():
            pltpu.sync_copy(acc.at[pl.ds(ch * _ZR, _ZR)], zbuf)
            pltpu.sync_copy(zbuf, out_hbm.at[c, pl.ds(ch * _ZR, _ZR)])

        return carry

    lax.fori_loop(0, pl.cdiv(nch, _NS), out_body, 0)


def _make_agg_kernel(N, D, nc):
    mesh = plsc.VectorSubcoreMesh(core_axis_name="c", subcore_axis_name="s")
    return pl.kernel(
        functools.partial(_agg_body, N, nc),
        mesh=mesh,
        out_type=jax.ShapeDtypeStruct((_NC, N, D), jnp.float32),
        scratch_types=[
            pltpu.VMEM((nc, _K), jnp.int32),
            pltpu.VMEM((_G, _K), jnp.int32),
            pltpu.VMEM((_G, _K), jnp.int32),
            pltpu.VMEM((_K, D), jnp.float32),
            pltpu.VMEM((_K, D), jnp.float32),
            pltpu.VMEM((_ZR, D), jnp.float32),
            pltpu.VMEM_SHARED((N, D), jnp.float32),
            pltpu.SemaphoreType.DMA,
            pltpu.SemaphoreType.DMA,
            pltpu.SemaphoreType.DMA,
            pltpu.SemaphoreType.DMA,
        ],
    )


def _w2_body(betas, x_ref, w2_ref, xw2_ref):
    x0 = jnp.maximum(x_ref[...], 0.0)
    for i, b in enumerate(betas):
        m = jnp.dot(x0, w2_ref[i], preferred_element_type=jnp.float32)
        xw2_ref[i] = ALPHA * ((1.0 - b) * x0 + b * m)


def _make_w2(N, D, Lw, betas, R):
    return pl.pallas_call(
        functools.partial(_w2_body, betas),
        grid=(N // R,),
        in_specs=[
            pl.BlockSpec((R, D), lambda i: (i, 0)),
            pl.BlockSpec((Lw, D, D), lambda i: (0, 0, 0)),
        ],
        out_specs=[pl.BlockSpec((Lw, R, D), lambda i: (0, i, 0))],
        out_shape=[jax.ShapeDtypeStruct((Lw, N, D), jnp.float32)],
    )


def _pre_body(x_ref, degp_ref, g0_ref, dinv_ref):
    x0 = jnp.maximum(x_ref[...], 0.0)
    deg = degp_ref[0][:, 0:1] + degp_ref[1][:, 0:1] + 1.0
    dinv = lax.rsqrt(deg)
    dinv_ref[...] = jnp.broadcast_to(dinv, x0.shape)
    g0_ref[...] = dinv * x0


def _make_pre(N, D, R):
    return pl.pallas_call(
        _pre_body,
        grid=(N // R,),
        in_specs=[
            pl.BlockSpec((R, D), lambda i: (i, 0)),
            pl.BlockSpec((_NC, R, D), lambda i: (0, i, 0)),
        ],
        out_specs=[
            pl.BlockSpec((R, D), lambda i: (i, 0)),
            pl.BlockSpec((R, D), lambda i: (i, 0)),
        ],
        out_shape=[
            jax.ShapeDtypeStruct((N, D), jnp.float32),
            jax.ShapeDtypeStruct((N, D), jnp.float32),
        ],
    )


def _layer_body(beta, p_ref, g_ref, dinv_ref, xw2_ref, w1_ref, h_ref, g2_ref):
    dinv = dinv_ref[...]
    t = dinv * (p_ref[0] + p_ref[1] + g_ref[...])
    m = jnp.dot(t, w1_ref[...], preferred_element_type=jnp.float32)
    u = (1.0 - ALPHA) * ((1.0 - beta) * t + beta * m) + xw2_ref[...]
    h = jnp.maximum(u, 0.0)
    h_ref[...] = h
    g2_ref[...] = dinv * h


def _make_layer(N, D, beta, R):
    return pl.pallas_call(
        functools.partial(_layer_body, beta),
        grid=(N // R,),
        in_specs=[
            pl.BlockSpec((_NC, R, D), lambda i: (0, i, 0)),
            pl.BlockSpec((R, D), lambda i: (i, 0)),
            pl.BlockSpec((R, D), lambda i: (i, 0)),
            pl.BlockSpec((R, D), lambda i: (i, 0)),
            pl.BlockSpec((D, D), lambda i: (0, 0)),
        ],
        out_specs=[
            pl.BlockSpec((R, D), lambda i: (i, 0)),
            pl.BlockSpec((R, D), lambda i: (i, 0)),
        ],
        out_shape=[
            jax.ShapeDtypeStruct((N, D), jnp.float32),
            jax.ShapeDtypeStruct((N, D), jnp.float32),
        ],
    )


@jax.jit
def kernel(x, edge_index, W1, W2):
    N, D = x.shape
    E = edge_index.shape[1]
    Lw = W1.shape[0]
    betas = [math.log(THETA / (i + 1) + 1.0) for i in range(Lw)]
    R = 1000

    nc = (E // _NW) // _K
    nc_pad = pl.cdiv(nc, _G) * _G
    src3d = edge_index[0].reshape(_NW, nc, _K)
    dst3d = edge_index[1].reshape(_NW, nc, _K)
    dst3d = jnp.pad(dst3d, ((0, 0), (0, nc_pad - nc), (0, 0)))
    zer = jnp.zeros((_ZR, D), jnp.float32)
    onesk = jnp.ones((_K, D), jnp.float32)

    degp = _make_deg_kernel(N, D, nc, nc_pad)(dst3d, onesk, zer)
    (xw2,) = _make_w2(N, D, Lw, betas, R)(x, W2)
    g0, dinvb = _make_pre(N, D, R)(x, degp)

    agg = _make_agg_kernel(N, D, nc)
    g = g0
    h = None
    for i in range(Lw):
        part = agg(g, src3d, dst3d, zer)
        h, g = _make_layer(N, D, betas[i], R)(part, g, dinvb, xw2[i], W1[i])
    return h


# speedup vs baseline: 6.8300x; 1.0268x over previous
import functools
import math

import jax
import jax.numpy as jnp
from jax import lax
from jax.experimental import pallas as pl
from jax.experimental.pallas import tpu as pltpu
from jax.experimental.pallas import tpu_sc as plsc

ALPHA = 0.5
THETA = 1.0

_NC = 2
_NS = 16
_NW = _NC * _NS
_K = 100
_G = 16
_ZR = 40


def _deg_body(N, nc, dst_hbm, ones_hbm, zer_hbm, out_hbm, idx_d, onev, zbuf, acc):
    c = lax.axis_index("c")
    s = lax.axis_index("s")
    wid = c * _NS + s
    nch = N // _ZR

    pltpu.sync_copy(dst_hbm.at[wid], idx_d)
    pltpu.sync_copy(ones_hbm, onev)
    pltpu.sync_copy(zer_hbm, zbuf)

    def zc_body(m, carry):
        ch = m * _NS + s

        @pl.when(ch < nch)
        def _():
            pltpu.sync_copy(zbuf, acc.at[pl.ds(ch * _ZR, _ZR)])

        return carry

    lax.fori_loop(0, pl.cdiv(nch, _NS), zc_body, 0)
    plsc.subcore_barrier()

    def body(j, carry):
        pltpu.sync_copy(onev, acc.at[idx_d.at[j]], add=True)
        return carry

    lax.fori_loop(0, nc, body, 0)
    plsc.subcore_barrier()

    def out_body(m, carry):
        ch = m * _NS + s

        @pl.when(ch < nch)
        def _():
            pltpu.sync_copy(acc.at[pl.ds(ch * _ZR, _ZR)], zbuf)
            pltpu.sync_copy(zbuf, out_hbm.at[c, pl.ds(ch * _ZR, _ZR)])

        return carry

    lax.fori_loop(0, pl.cdiv(nch, _NS), out_body, 0)


def _make_deg_kernel(N, D, nc, nc_pad):
    mesh = plsc.VectorSubcoreMesh(core_axis_name="c", subcore_axis_name="s")
    return pl.kernel(
        functools.partial(_deg_body, N, nc),
        mesh=mesh,
        out_type=jax.ShapeDtypeStruct((_NC, N, D), jnp.float32),
        scratch_types=[
            pltpu.VMEM((nc_pad, _K), jnp.int32),
            pltpu.VMEM((_K, D), jnp.float32),
            pltpu.VMEM((_ZR, D), jnp.float32),
            pltpu.VMEM_SHARED((N, D), jnp.float32),
        ],
    )


def _agg_body(N, nc, g_hbm, src_hbm, dst_hbm, zer_hbm, out_hbm,
              idx_s, dg0, dg1, rows0, rows1, zbuf, acc,
              sg0, sg1, sd0, sd1):
    c = lax.axis_index("c")
    s = lax.axis_index("s")
    wid = c * _NS + s
    nfull = nc // _G
    rem = nc - nfull * _G
    assert nfull % 2 == 0 and rem % 2 == 0 and rem < _G

    pltpu.sync_copy(src_hbm.at[wid], idx_s)
    pltpu.async_copy(dst_hbm.at[wid, pl.ds(0, _G)], dg0, sd0)
    pltpu.async_copy(dst_hbm.at[wid, pl.ds(_G, _G)], dg1, sd1)

    pltpu.sync_copy(zer_hbm, zbuf)
    nch = N // _ZR

    def zc_issue(m, carry):
        ch = m * _NS + s

        @pl.when(ch < nch)
        def _():
            pltpu.async_copy(zbuf, acc.at[pl.ds(ch * _ZR, _ZR)], sg0)

        return carry

    lax.fori_loop(0, pl.cdiv(nch, _NS), zc_issue, 0)

    def zc_drain(m, carry):
        ch = m * _NS + s

        @pl.when(ch < nch)
        def _():
            pltpu.make_async_copy(zbuf, acc.at[pl.ds(0, _ZR)], sg0).wait()

        return carry

    lax.fori_loop(0, pl.cdiv(nch, _NS), zc_drain, 0)
    plsc.subcore_barrier()

    pltpu.async_copy(g_hbm.at[idx_s.at[0]], rows0, sg0)
    pltpu.async_copy(g_hbm.at[idx_s.at[1]], rows1, sg1)

    def run_group(gbase, dg, sd, next_base):
        pltpu.make_async_copy(dst_hbm.at[wid, pl.ds(0, _G)], dg, sd).wait()

        def pair(l2, carry):
            for t, (rb, sgb) in ((0, (rows0, sg0)), (1, (rows1, sg1))):
                l = 2 * l2 + t
                j = gbase + l
                pltpu.make_async_copy(g_hbm.at[idx_s.at[0]], rb, sgb).wait()
                pltpu.sync_copy(rb, acc.at[dg.at[l]], add=True)

                @pl.when(j + 2 < nc)
                def _():
                    pltpu.async_copy(g_hbm.at[idx_s.at[j + 2]], rb, sgb)

            return carry

        lax.fori_loop(0, _G // 2, pair, 0)

        @pl.when(next_base < nc)
        def _():
            pltpu.async_copy(dst_hbm.at[wid, pl.ds(next_base, _G)], dg, sd)

    def groups(m, carry):
        gbase = (2 * m) * _G
        run_group(gbase, dg0, sd0, gbase + 2 * _G)
        run_group(gbase + _G, dg1, sd1, gbase + 3 * _G)
        return carry

    lax.fori_loop(0, nfull // 2, groups, 0)

    if rem:
        pltpu.make_async_copy(dst_hbm.at[wid, pl.ds(0, _G)], dg0, sd0).wait()
        for l in range(rem):
            j = nfull * _G + l
            rb, sgb = (rows0, sg0) if l % 2 == 0 else (rows1, sg1)
            pltpu.make_async_copy(g_hbm.at[idx_s.at[0]], rb, sgb).wait()
            pltpu.sync_copy(rb, acc.at[dg0.at[l]], add=True)
            if l + 2 < rem:
                pltpu.async_copy(g_hbm.at[idx_s.at[j + 2]], rb, sgb)
    plsc.subcore_barrier()

    def out_chunk(m, rb, sgb):
        ch = m * _NS + s

        @pl.when(ch < nch)
        def _():
            @pl.when(m >= 2)
            def _():
                pltpu.make_async_copy(
                    rb.at[pl.ds(0, _ZR)], out_hbm.at[c, pl.ds(0, _ZR)], sgb
                ).wait()

            pltpu.sync_copy(acc.at[pl.ds(ch * _ZR, _ZR)], rb.at[pl.ds(0, _ZR)])
            pltpu.async_copy(
                rb.at[pl.ds(0, _ZR)], out_hbm.at[c, pl.ds(ch * _ZR, _ZR)], sgb)

    def out_body(m2, carry):
        out_chunk(2 * m2, rows0, sg0)
        out_chunk(2 * m2 + 1, rows1, sg1)
        return carry

    lax.fori_loop(0, pl.cdiv(pl.cdiv(nch, _NS), 2), out_body, 0)
    pltpu.make_async_copy(
        rows0.at[pl.ds(0, _ZR)], out_hbm.at[c, pl.ds(0, _ZR)], sg0).wait()
    pltpu.make_async_copy(
        rows1.at[pl.ds(0, _ZR)], out_hbm.at[c, pl.ds(0, _ZR)], sg1).wait()


def _make_agg_kernel(N, D, nc):
    mesh = plsc.VectorSubcoreMesh(core_axis_name="c", subcore_axis_name="s")
    return pl.kernel(
        functools.partial(_agg_body, N, nc),
        mesh=mesh,
        out_type=jax.ShapeDtypeStruct((_NC, N, D), jnp.float32),
        scratch_types=[
            pltpu.VMEM((nc, _K), jnp.int32),
            pltpu.VMEM((_G, _K), jnp.int32),
            pltpu.VMEM((_G, _K), jnp.int32),
            pltpu.VMEM((_K, D), jnp.float32),
            pltpu.VMEM((_K, D), jnp.float32),
            pltpu.VMEM((_ZR, D), jnp.float32),
            pltpu.VMEM_SHARED((N, D), jnp.float32),
            pltpu.SemaphoreType.DMA,
            pltpu.SemaphoreType.DMA,
            pltpu.SemaphoreType.DMA,
            pltpu.SemaphoreType.DMA,
        ],
    )


def _w2_body(betas, x_ref, w2_ref, xw2_ref):
    x0 = jnp.maximum(x_ref[...], 0.0)
    for i, b in enumerate(betas):
        m = jnp.dot(x0, w2_ref[i], preferred_element_type=jnp.float32)
        xw2_ref[i] = ALPHA * ((1.0 - b) * x0 + b * m)


def _make_w2(N, D, Lw, betas, R):
    return pl.pallas_call(
        functools.partial(_w2_body, betas),
        grid=(N // R,),
        in_specs=[
            pl.BlockSpec((R, D), lambda i: (i, 0)),
            pl.BlockSpec((Lw, D, D), lambda i: (0, 0, 0)),
        ],
        out_specs=[pl.BlockSpec((Lw, R, D), lambda i: (0, i, 0))],
        out_shape=[jax.ShapeDtypeStruct((Lw, N, D), jnp.float32)],
    )


def _pre_body(x_ref, degp_ref, g0_ref, dinv_ref):
    x0 = jnp.maximum(x_ref[...], 0.0)
    deg = degp_ref[0][:, 0:1] + degp_ref[1][:, 0:1] + 1.0
    dinv = lax.rsqrt(deg)
    dinv_ref[...] = jnp.broadcast_to(dinv, x0.shape)
    g0_ref[...] = dinv * x0


def _make_pre(N, D, R):
    return pl.pallas_call(
        _pre_body,
        grid=(N // R,),
        in_specs=[
            pl.BlockSpec((R, D), lambda i: (i, 0)),
            pl.BlockSpec((_NC, R, D), lambda i: (0, i, 0)),
        ],
        out_specs=[
            pl.BlockSpec((R, D), lambda i: (i, 0)),
            pl.BlockSpec((R, D), lambda i: (i, 0)),
        ],
        out_shape=[
            jax.ShapeDtypeStruct((N, D), jnp.float32),
            jax.ShapeDtypeStruct((N, D), jnp.float32),
        ],
    )


def _layer_body(beta, p_ref, g_ref, dinv_ref, xw2_ref, w1_ref, h_ref, g2_ref):
    dinv = dinv_ref[...]
    t = dinv * (p_ref[0] + p_ref[1] + g_ref[...])
    m = jnp.dot(t, w1_ref[...], preferred_element_type=jnp.float32)
    u = (1.0 - ALPHA) * ((1.0 - beta) * t + beta * m) + xw2_ref[...]
    h = jnp.maximum(u, 0.0)
    h_ref[...] = h
    g2_ref[...] = dinv * h


def _make_layer(N, D, beta, R):
    return pl.pallas_call(
        functools.partial(_layer_body, beta),
        grid=(N // R,),
        in_specs=[
            pl.BlockSpec((_NC, R, D), lambda i: (0, i, 0)),
            pl.BlockSpec((R, D), lambda i: (i, 0)),
            pl.BlockSpec((R, D), lambda i: (i, 0)),
            pl.BlockSpec((R, D), lambda i: (i, 0)),
            pl.BlockSpec((D, D), lambda i: (0, 0)),
        ],
        out_specs=[
            pl.BlockSpec((R, D), lambda i: (i, 0)),
            pl.BlockSpec((R, D), lambda i: (i, 0)),
        ],
        out_shape=[
            jax.ShapeDtypeStruct((N, D), jnp.float32),
            jax.ShapeDtypeStruct((N, D), jnp.float32),
        ],
    )


@jax.jit
def kernel(x, edge_index, W1, W2):
    N, D = x.shape
    E = edge_index.shape[1]
    Lw = W1.shape[0]
    betas = [math.log(THETA / (i + 1) + 1.0) for i in range(Lw)]
    R = 1000

    nc = (E // _NW) // _K
    nc_pad = pl.cdiv(nc, _G) * _G
    src3d = edge_index[0].reshape(_NW, nc, _K)
    dst3d = edge_index[1].reshape(_NW, nc, _K)
    dst3d = jnp.pad(dst3d, ((0, 0), (0, nc_pad - nc), (0, 0)))
    zer = jnp.zeros((_ZR, D), jnp.float32)
    onesk = jnp.ones((_K, D), jnp.float32)

    degp = _make_deg_kernel(N, D, nc, nc_pad)(dst3d, onesk, zer)
    (xw2,) = _make_w2(N, D, Lw, betas, R)(x, W2)
    g0, dinvb = _make_pre(N, D, R)(x, degp)

    agg = _make_agg_kernel(N, D, nc)
    g = g0
    h = None
    for i in range(Lw):
        part = agg(g, src3d, dst3d, zer)
        h, g = _make_layer(N, D, betas[i], R)(part, g, dinvb, xw2[i], W1[i])
    return h


# speedup vs baseline: 6.8760x; 1.0067x over previous
import functools
import math

import jax
import jax.numpy as jnp
from jax import lax
from jax.experimental import pallas as pl
from jax.experimental.pallas import tpu as pltpu
from jax.experimental.pallas import tpu_sc as plsc

ALPHA = 0.5
THETA = 1.0

_NC = 2
_NS = 16
_NW = _NC * _NS
_K = 100
_G = 16
_ZR = 40


def _deg_body(N, nc, dst_hbm, ones_hbm, zer_hbm, out_hbm, idx_d, onev, zbuf, acc,
              s0, s1):
    c = lax.axis_index("c")
    s = lax.axis_index("s")
    wid = c * _NS + s
    nch = N // _ZR

    pltpu.sync_copy(dst_hbm.at[wid], idx_d)
    pltpu.sync_copy(ones_hbm, onev)
    pltpu.sync_copy(zer_hbm, zbuf)

    def zc_issue(m, carry):
        ch = m * _NS + s

        @pl.when(ch < nch)
        def _():
            pltpu.async_copy(zbuf, acc.at[pl.ds(ch * _ZR, _ZR)], s0)

        return carry

    lax.fori_loop(0, pl.cdiv(nch, _NS), zc_issue, 0)

    def zc_drain(m, carry):
        ch = m * _NS + s

        @pl.when(ch < nch)
        def _():
            pltpu.make_async_copy(zbuf, acc.at[pl.ds(0, _ZR)], s0).wait()

        return carry

    lax.fori_loop(0, pl.cdiv(nch, _NS), zc_drain, 0)
    plsc.subcore_barrier()

    def body(j, carry):
        pltpu.sync_copy(onev, acc.at[idx_d.at[j]], add=True)
        return carry

    lax.fori_loop(0, nc, body, 0)
    plsc.subcore_barrier()

    def out_chunk(m, rb, sem):
        ch = m * _NS + s

        @pl.when(ch < nch)
        def _():
            @pl.when(m >= 2)
            def _():
                pltpu.make_async_copy(
                    rb.at[pl.ds(0, _ZR)], out_hbm.at[c, pl.ds(0, _ZR)], sem
                ).wait()

            pltpu.sync_copy(acc.at[pl.ds(ch * _ZR, _ZR)], rb.at[pl.ds(0, _ZR)])
            pltpu.async_copy(
                rb.at[pl.ds(0, _ZR)], out_hbm.at[c, pl.ds(ch * _ZR, _ZR)], sem)

    def out_body(m2, carry):
        out_chunk(2 * m2, zbuf, s0)
        out_chunk(2 * m2 + 1, onev, s1)
        return carry

    lax.fori_loop(0, pl.cdiv(pl.cdiv(nch, _NS), 2), out_body, 0)
    pltpu.make_async_copy(
        zbuf.at[pl.ds(0, _ZR)], out_hbm.at[c, pl.ds(0, _ZR)], s0).wait()
    pltpu.make_async_copy(
        onev.at[pl.ds(0, _ZR)], out_hbm.at[c, pl.ds(0, _ZR)], s1).wait()


def _make_deg_kernel(N, D, nc, nc_pad):
    mesh = plsc.VectorSubcoreMesh(core_axis_name="c", subcore_axis_name="s")
    return pl.kernel(
        functools.partial(_deg_body, N, nc),
        mesh=mesh,
        out_type=jax.ShapeDtypeStruct((_NC, N, D), jnp.float32),
        scratch_types=[
            pltpu.VMEM((nc_pad, _K), jnp.int32),
            pltpu.VMEM((_K, D), jnp.float32),
            pltpu.VMEM((_ZR, D), jnp.float32),
            pltpu.VMEM_SHARED((N, D), jnp.float32),
            pltpu.SemaphoreType.DMA,
            pltpu.SemaphoreType.DMA,
        ],
    )


def _agg_body(N, nc, g_hbm, src_hbm, dst_hbm, zer_hbm, out_hbm,
              idx_s, dg0, dg1, rows0, rows1, zbuf, acc,
              sg0, sg1, sd0, sd1):
    c = lax.axis_index("c")
    s = lax.axis_index("s")
    wid = c * _NS + s
    nfull = nc // _G
    rem = nc - nfull * _G
    assert nfull % 2 == 0 and rem % 2 == 0 and rem < _G

    pltpu.sync_copy(src_hbm.at[wid], idx_s)
    pltpu.async_copy(dst_hbm.at[wid, pl.ds(0, _G)], dg0, sd0)
    pltpu.async_copy(dst_hbm.at[wid, pl.ds(_G, _G)], dg1, sd1)

    pltpu.sync_copy(zer_hbm, zbuf)
    nch = N // _ZR

    def zc_issue(m, carry):
        ch = m * _NS + s

        @pl.when(ch < nch)
        def _():
            pltpu.async_copy(zbuf, acc.at[pl.ds(ch * _ZR, _ZR)], sg0)

        return carry

    lax.fori_loop(0, pl.cdiv(nch, _NS), zc_issue, 0)

    def zc_drain(m, carry):
        ch = m * _NS + s

        @pl.when(ch < nch)
        def _():
            pltpu.make_async_copy(zbuf, acc.at[pl.ds(0, _ZR)], sg0).wait()

        return carry

    lax.fori_loop(0, pl.cdiv(nch, _NS), zc_drain, 0)
    plsc.subcore_barrier()

    pltpu.async_copy(g_hbm.at[idx_s.at[0]], rows0, sg0)
    pltpu.async_copy(g_hbm.at[idx_s.at[1]], rows1, sg1)

    def run_group(gbase, dg, sd, next_base):
        pltpu.make_async_copy(dst_hbm.at[wid, pl.ds(0, _G)], dg, sd).wait()

        def pair(l2, carry):
            for t, (rb, sgb) in ((0, (rows0, sg0)), (1, (rows1, sg1))):
                l = 2 * l2 + t
                j = gbase + l
                pltpu.make_async_copy(g_hbm.at[idx_s.at[0]], rb, sgb).wait()
                pltpu.sync_copy(rb, acc.at[dg.at[l]], add=True)

                @pl.when(j + 2 < nc)
                def _():
                    pltpu.async_copy(g_hbm.at[idx_s.at[j + 2]], rb, sgb)

            return carry

        lax.fori_loop(0, _G // 2, pair, 0)

        @pl.when(next_base < nc)
        def _():
            pltpu.async_copy(dst_hbm.at[wid, pl.ds(next_base, _G)], dg, sd)

    def groups(m, carry):
        gbase = (2 * m) * _G
        run_group(gbase, dg0, sd0, gbase + 2 * _G)
        run_group(gbase + _G, dg1, sd1, gbase + 3 * _G)
        return carry

    lax.fori_loop(0, nfull // 2, groups, 0)

    if rem:
        pltpu.make_async_copy(dst_hbm.at[wid, pl.ds(0, _G)], dg0, sd0).wait()
        for l in range(rem):
            j = nfull * _G + l
            rb, sgb = (rows0, sg0) if l % 2 == 0 else (rows1, sg1)
            pltpu.make_async_copy(g_hbm.at[idx_s.at[0]], rb, sgb).wait()
            pltpu.sync_copy(rb, acc.at[dg0.at[l]], add=True)
            if l + 2 < rem:
                pltpu.async_copy(g_hbm.at[idx_s.at[j + 2]], rb, sgb)
    plsc.subcore_barrier()

    def out_chunk(m, rb, sgb):
        ch = m * _NS + s

        @pl.when(ch < nch)
        def _():
            @pl.when(m >= 2)
            def _():
                pltpu.make_async_copy(
                    rb.at[pl.ds(0, _ZR)], out_hbm.at[c, pl.ds(0, _ZR)], sgb
                ).wait()

            pltpu.sync_copy(acc.at[pl.ds(ch * _ZR, _ZR)], rb.at[pl.ds(0, _ZR)])
            pltpu.async_copy(
                rb.at[pl.ds(0, _ZR)], out_hbm.at[c, pl.ds(ch * _ZR, _ZR)], sgb)

    def out_body(m2, carry):
        out_chunk(2 * m2, rows0, sg0)
        out_chunk(2 * m2 + 1, rows1, sg1)
        return carry

    lax.fori_loop(0, pl.cdiv(pl.cdiv(nch, _NS), 2), out_body, 0)
    pltpu.make_async_copy(
        rows0.at[pl.ds(0, _ZR)], out_hbm.at[c, pl.ds(0, _ZR)], sg0).wait()
    pltpu.make_async_copy(
        rows1.at[pl.ds(0, _ZR)], out_hbm.at[c, pl.ds(0, _ZR)], sg1).wait()


def _make_agg_kernel(N, D, nc):
    mesh = plsc.VectorSubcoreMesh(core_axis_name="c", subcore_axis_name="s")
    return pl.kernel(
        functools.partial(_agg_body, N, nc),
        mesh=mesh,
        out_type=jax.ShapeDtypeStruct((_NC, N, D), jnp.float32),
        scratch_types=[
            pltpu.VMEM((nc, _K), jnp.int32),
            pltpu.VMEM((_G, _K), jnp.int32),
            pltpu.VMEM((_G, _K), jnp.int32),
            pltpu.VMEM((_K, D), jnp.float32),
            pltpu.VMEM((_K, D), jnp.float32),
            pltpu.VMEM((_ZR, D), jnp.float32),
            pltpu.VMEM_SHARED((N, D), jnp.float32),
            pltpu.SemaphoreType.DMA,
            pltpu.SemaphoreType.DMA,
            pltpu.SemaphoreType.DMA,
            pltpu.SemaphoreType.DMA,
        ],
    )


def _w2_body(betas, x_ref, w2_ref, xw2_ref):
    x0 = jnp.maximum(x_ref[...], 0.0)
    for i, b in enumerate(betas):
        m = jnp.dot(x0, w2_ref[i], preferred_element_type=jnp.float32)
        xw2_ref[i] = ALPHA * ((1.0 - b) * x0 + b * m)


def _make_w2(N, D, Lw, betas, R):
    return pl.pallas_call(
        functools.partial(_w2_body, betas),
        grid=(N // R,),
        in_specs=[
            pl.BlockSpec((R, D), lambda i: (i, 0)),
            pl.BlockSpec((Lw, D, D), lambda i: (0, 0, 0)),
        ],
        out_specs=[pl.BlockSpec((Lw, R, D), lambda i: (0, i, 0))],
        out_shape=[jax.ShapeDtypeStruct((Lw, N, D), jnp.float32)],
    )


def _pre_body(x_ref, degp_ref, g0_ref, dinv_ref):
    x0 = jnp.maximum(x_ref[...], 0.0)
    deg = degp_ref[0][:, 0:1] + degp_ref[1][:, 0:1] + 1.0
    dinv = lax.rsqrt(deg)
    dinv_ref[...] = jnp.broadcast_to(dinv, x0.shape)
    g0_ref[...] = dinv * x0


def _make_pre(N, D, R):
    return pl.pallas_call(
        _pre_body,
        grid=(N // R,),
        in_specs=[
            pl.BlockSpec((R, D), lambda i: (i, 0)),
            pl.BlockSpec((_NC, R, D), lambda i: (0, i, 0)),
        ],
        out_specs=[
            pl.BlockSpec((R, D), lambda i: (i, 0)),
            pl.BlockSpec((R, D), lambda i: (i, 0)),
        ],
        out_shape=[
            jax.ShapeDtypeStruct((N, D), jnp.float32),
            jax.ShapeDtypeStruct((N, D), jnp.float32),
        ],
    )


def _layer_body(beta, p_ref, g_ref, dinv_ref, xw2_ref, w1_ref, h_ref, g2_ref):
    dinv = dinv_ref[...]
    t = dinv * (p_ref[0] + p_ref[1] + g_ref[...])
    m = jnp.dot(t, w1_ref[...], preferred_element_type=jnp.float32)
    u = (1.0 - ALPHA) * ((1.0 - beta) * t + beta * m) + xw2_ref[...]
    h = jnp.maximum(u, 0.0)
    h_ref[...] = h
    g2_ref[...] = dinv * h


def _make_layer(N, D, beta, R):
    return pl.pallas_call(
        functools.partial(_layer_body, beta),
        grid=(N // R,),
        in_specs=[
            pl.BlockSpec((_NC, R, D), lambda i: (0, i, 0)),
            pl.BlockSpec((R, D), lambda i: (i, 0)),
            pl.BlockSpec((R, D), lambda i: (i, 0)),
            pl.BlockSpec((R, D), lambda i: (i, 0)),
            pl.BlockSpec((D, D), lambda i: (0, 0)),
        ],
        out_specs=[
            pl.BlockSpec((R, D), lambda i: (i, 0)),
            pl.BlockSpec((R, D), lambda i: (i, 0)),
        ],
        out_shape=[
            jax.ShapeDtypeStruct((N, D), jnp.float32),
            jax.ShapeDtypeStruct((N, D), jnp.float32),
        ],
    )


@jax.jit
def kernel(x, edge_index, W1, W2):
    N, D = x.shape
    E = edge_index.shape[1]
    Lw = W1.shape[0]
    betas = [math.log(THETA / (i + 1) + 1.0) for i in range(Lw)]
    R = 1000

    nc = (E // _NW) // _K
    nc_pad = pl.cdiv(nc, _G) * _G
    src3d = edge_index[0].reshape(_NW, nc, _K)
    dst3d = edge_index[1].reshape(_NW, nc, _K)
    dst3d = jnp.pad(dst3d, ((0, 0), (0, nc_pad - nc), (0, 0)))
    zer = jnp.zeros((_ZR, D), jnp.float32)
    onesk = jnp.ones((_K, D), jnp.float32)

    degp = _make_deg_kernel(N, D, nc, nc_pad)(dst3d, onesk, zer)
    (xw2,) = _make_w2(N, D, Lw, betas, R)(x, W2)
    g0, dinvb = _make_pre(N, D, R)(x, degp)

    agg = _make_agg_kernel(N, D, nc)
    g = g0
    h = None
    for i in range(Lw):
        part = agg(g, src3d, dst3d, zer)
        h, g = _make_layer(N, D, betas[i], R)(part, g, dinvb, xw2[i], W1[i])
    return h
